# baseline (device time: 138098 ns/iter reference)
import functools

import jax
import jax.numpy as jnp
from jax import lax
from jax.experimental import pallas as pl
from jax.experimental.pallas import tpu as pltpu

N_DEV = 8
N_EXP = 32

MASKS = (1, 3, 4)
ORDERS = ((1, 3, 4), (3, 4, 1), (4, 1, 3))
PACK = (
    ((0, 0, 1024), (1, 0, 384)),
    ((1, 384, 1024), (2, 0, 768)),
    ((2, 768, 1024), (3, 0, 1024)),
)
NCOLS = tuple(sum(ce - cs for _, cs, ce in p) for p in PACK)


def kernel(x, router_W, route_idx, expert_W):
    n_tok, d_model = x.shape
    e_loc, _, d_hidden = expert_W.shape

    def body(x_ref, router_ref, idx_ref, w_ref, out_ref,
             g0_ref, g1_ref, g2_ref, send_sems, recv_sems):
        my = lax.axis_index("i")
        gs = (g0_ref, g1_ref, g2_ref)

        barrier = pltpu.get_barrier_semaphore()
        for mask in MASKS:
            pl.semaphore_signal(barrier, inc=1, device_id=(my ^ mask,),
                                device_id_type=pl.DeviceIdType.MESH)
        pl.semaphore_wait(barrier, 3)

        descs = [[None] * 7 for _ in range(3)]

        def issue(a, step, pairs):
            mask = ORDERS[a][step]
            partner = my ^ mask
            for g, fi in pairs:
                rdma = pltpu.make_async_remote_copy(
                    src_ref=gs[a].at[g],
                    dst_ref=gs[a].at[g ^ mask],
                    send_sem=send_sems.at[a, fi],
                    recv_sem=recv_sems.at[a, fi],
                    device_id=(partner,),
                    device_id_type=pl.DeviceIdType.MESH,
                )
                rdma.start()
                descs[a][fi] = rdma

        for a in range(3):
            off = 0
            for j, cs, ce in PACK[a]:
                w = ce - cs
                gs[a][0, :, off:off + w] = (
                    w_ref[j, :, cs:ce].astype(jnp.bfloat16))
                off += w
            issue(a, 0, [(0, 0)])
        for a in range(3):
            issue(a, 1, [(0, 1)])
            issue(a, 2, [(0, 3)])

        x_f32 = x_ref[:, :]
        x_bf = x_f32.astype(jnp.bfloat16)

        scores = jnp.dot(x_bf, router_ref[:, :].astype(jnp.bfloat16),
                         preferred_element_type=jnp.float32)
        e0 = idx_ref[:, 0:1]
        e1 = idx_ref[:, 1:2]
        col = lax.broadcasted_iota(jnp.int32, (n_tok, N_EXP), 1)
        s0 = jnp.sum(jnp.where(col == e0, scores, 0.0), axis=1, keepdims=True)
        s1 = jnp.sum(jnp.where(col == e1, scores, 0.0), axis=1, keepdims=True)
        m = jnp.maximum(s0, s1)
        p0 = jnp.exp(s0 - m)
        p1 = jnp.exp(s1 - m)
        w0 = p0 / (p0 + p1)
        w1 = p1 / (p0 + p1)

        out_ref[...] = jnp.zeros((n_tok, d_hidden), jnp.float32)

        def compute(a, g):
            src_dev = my ^ g
            off = 0
            for j, cs, ce in PACK[a]:
                w = ce - cs
                eg = src_dev * e_loc + j
                w_e = (jnp.where(e0 == eg, w0, 0.0)
                       + jnp.where(e1 == eg, w1, 0.0))
                x_e = (w_e * x_f32).astype(jnp.bfloat16)
                y = jnp.dot(x_e, gs[a][g, :, off:off + w],
                            preferred_element_type=jnp.float32)
                out_ref[:, cs:ce] = out_ref[:, cs:ce] + y
                off += w

        for a in range(3):
            compute(a, 0)
        for a in range(3):
            descs[a][0].wait_recv()

        for a in range(3):
            m0 = ORDERS[a][0]
            issue(a, 1, [(m0, 2)])
            issue(a, 2, [(m0, 4)])
        for a in range(3):
            compute(a, ORDERS[a][0])
        for a in range(3):
            descs[a][1].wait_recv()
            descs[a][2].wait_recv()

        for a in range(3):
            m0, m1, _ = ORDERS[a]
            issue(a, 2, [(m1, 5), (m0 ^ m1, 6)])
        for a in range(3):
            m0, m1, _ = ORDERS[a]
            compute(a, m1)
            compute(a, m0 ^ m1)
        for a in range(3):
            descs[a][3].wait_recv()
        for a in range(3):
            compute(a, ORDERS[a][2])
        for a in range(3):
            descs[a][4].wait_recv()
        for a in range(3):
            m0, _, m2 = ORDERS[a]
            compute(a, m0 ^ m2)
        for a in range(3):
            descs[a][5].wait_recv()
        for a in range(3):
            _, m1, m2 = ORDERS[a]
            compute(a, m1 ^ m2)
        for a in range(3):
            descs[a][6].wait_recv()
        for a in range(3):
            m0, m1, m2 = ORDERS[a]
            compute(a, m0 ^ m1 ^ m2)

        for a in range(3):
            for fi in range(7):
                descs[a][fi].wait_send()

        @functools.partial(pl.run_scoped,
                           second_barrier=pltpu.SemaphoreType.REGULAR)
        def _(second_barrier):
            for mask in MASKS:
                pl.semaphore_signal(second_barrier, inc=1,
                                    device_id=(my ^ mask,),
                                    device_id_type=pl.DeviceIdType.MESH)
            pl.semaphore_wait(second_barrier, 3)

    return pl.pallas_call(
        body,
        out_shape=jax.ShapeDtypeStruct((n_tok, d_hidden), jnp.float32),
        in_specs=[pl.BlockSpec(memory_space=pltpu.VMEM)] * 4,
        out_specs=pl.BlockSpec(memory_space=pltpu.VMEM),
        scratch_shapes=[
            pltpu.VMEM((N_DEV, d_model, NCOLS[0]), jnp.bfloat16),
            pltpu.VMEM((N_DEV, d_model, NCOLS[1]), jnp.bfloat16),
            pltpu.VMEM((N_DEV, d_model, NCOLS[2]), jnp.bfloat16),
            pltpu.SemaphoreType.DMA((3, 7)),
            pltpu.SemaphoreType.DMA((3, 7)),
        ],
        compiler_params=pltpu.CompilerParams(
            collective_id=0,
            vmem_limit_bytes=64 * 1024 * 1024,
        ),
    )(x, router_W, route_idx, expert_W)


# device time: 135052 ns/iter; 1.0226x vs baseline; 1.0226x over previous
import functools

import jax
import jax.numpy as jnp
from jax import lax
from jax.experimental import pallas as pl
from jax.experimental.pallas import tpu as pltpu

N_DEV = 8
N_EXP = 32

MASKS = (1, 3, 4)
ORDERS = ((1, 3, 4), (3, 4, 1), (4, 1, 3))
PACK = (
    ((0, 0, 1024), (1, 0, 384)),
    ((1, 384, 1024), (2, 0, 768)),
    ((2, 768, 1024), (3, 0, 1024)),
)
NCOLS = tuple(sum(ce - cs for _, cs, ce in p) for p in PACK)


def kernel(x, router_W, route_idx, expert_W):
    n_tok, d_model = x.shape
    e_loc, _, d_hidden = expert_W.shape

    def body(x_ref, router_ref, idx_ref, w_ref, out_ref,
             g0_ref, g1_ref, g2_ref, send_sems, recv_sems):
        my = lax.axis_index("i")
        gs = (g0_ref, g1_ref, g2_ref)

        barrier = pltpu.get_barrier_semaphore()
        for mask in MASKS:
            pl.semaphore_signal(barrier, inc=1, device_id=(my ^ mask,),
                                device_id_type=pl.DeviceIdType.MESH)
        pl.semaphore_wait(barrier, 3)

        descs = [[None] * 7 for _ in range(3)]

        def issue(a, step, pairs):
            mask = ORDERS[a][step]
            partner = my ^ mask
            for g, fi in pairs:
                rdma = pltpu.make_async_remote_copy(
                    src_ref=gs[a].at[g],
                    dst_ref=gs[a].at[g ^ mask],
                    send_sem=send_sems.at[a, fi],
                    recv_sem=recv_sems.at[a, fi],
                    device_id=(partner,),
                    device_id_type=pl.DeviceIdType.MESH,
                )
                rdma.start()
                descs[a][fi] = rdma

        for a in range(3):
            off = 0
            for j, cs, ce in PACK[a]:
                w = ce - cs
                gs[a][0, :, off:off + w] = (
                    w_ref[j, :, cs:ce].astype(jnp.bfloat16))
                off += w
            issue(a, 0, [(0, 0)])
        for a in range(3):
            issue(a, 1, [(0, 1)])
            issue(a, 2, [(0, 3)])

        x_bf = x_ref[:, :].astype(jnp.bfloat16)

        scores = jnp.dot(x_bf, router_ref[:, :].astype(jnp.bfloat16),
                         preferred_element_type=jnp.float32)
        e0 = idx_ref[:, 0:1]
        e1 = idx_ref[:, 1:2]
        col = lax.broadcasted_iota(jnp.int32, (n_tok, N_EXP), 1)
        s0 = jnp.sum(jnp.where(col == e0, scores, 0.0), axis=1, keepdims=True)
        s1 = jnp.sum(jnp.where(col == e1, scores, 0.0), axis=1, keepdims=True)
        m = jnp.maximum(s0, s1)
        p0 = jnp.exp(s0 - m)
        p1 = jnp.exp(s1 - m)
        w0 = p0 / (p0 + p1)
        w1 = p1 / (p0 + p1)

        out_ref[...] = jnp.zeros((n_tok, d_hidden), jnp.float32)

        def compute(a, g):
            src_dev = my ^ g
            off = 0
            for j, cs, ce in PACK[a]:
                w = ce - cs
                eg = src_dev * e_loc + j
                w_e = (jnp.where(e0 == eg, w0, 0.0)
                       + jnp.where(e1 == eg, w1, 0.0))
                y = jnp.dot(x_bf, gs[a][g, :, off:off + w],
                            preferred_element_type=jnp.float32)
                out_ref[:, cs:ce] = out_ref[:, cs:ce] + w_e * y
                off += w

        for a in range(3):
            compute(a, 0)
        for a in range(3):
            descs[a][0].wait_recv()

        for a in range(3):
            m0 = ORDERS[a][0]
            issue(a, 1, [(m0, 2)])
            issue(a, 2, [(m0, 4)])
        for a in range(3):
            compute(a, ORDERS[a][0])
        for a in range(3):
            descs[a][1].wait_recv()
            descs[a][2].wait_recv()

        for a in range(3):
            m0, m1, _ = ORDERS[a]
            issue(a, 2, [(m1, 5), (m0 ^ m1, 6)])
        for a in range(3):
            m0, m1, _ = ORDERS[a]
            compute(a, m1)
            compute(a, m0 ^ m1)
        for a in range(3):
            descs[a][3].wait_recv()
        for a in range(3):
            compute(a, ORDERS[a][2])
        for a in range(3):
            descs[a][4].wait_recv()
        for a in range(3):
            m0, _, m2 = ORDERS[a]
            compute(a, m0 ^ m2)
        for a in range(3):
            descs[a][5].wait_recv()
        for a in range(3):
            _, m1, m2 = ORDERS[a]
            compute(a, m1 ^ m2)
        for a in range(3):
            descs[a][6].wait_recv()
        for a in range(3):
            m0, m1, m2 = ORDERS[a]
            compute(a, m0 ^ m1 ^ m2)

        for a in range(3):
            for fi in range(7):
                descs[a][fi].wait_send()

        @functools.partial(pl.run_scoped,
                           second_barrier=pltpu.SemaphoreType.REGULAR)
        def _(second_barrier):
            for mask in MASKS:
                pl.semaphore_signal(second_barrier, inc=1,
                                    device_id=(my ^ mask,),
                                    device_id_type=pl.DeviceIdType.MESH)
            pl.semaphore_wait(second_barrier, 3)

    return pl.pallas_call(
        body,
        out_shape=jax.ShapeDtypeStruct((n_tok, d_hidden), jnp.float32),
        in_specs=[pl.BlockSpec(memory_space=pltpu.VMEM)] * 4,
        out_specs=pl.BlockSpec(memory_space=pltpu.VMEM),
        scratch_shapes=[
            pltpu.VMEM((N_DEV, d_model, NCOLS[0]), jnp.bfloat16),
            pltpu.VMEM((N_DEV, d_model, NCOLS[1]), jnp.bfloat16),
            pltpu.VMEM((N_DEV, d_model, NCOLS[2]), jnp.bfloat16),
            pltpu.SemaphoreType.DMA((3, 7)),
            pltpu.SemaphoreType.DMA((3, 7)),
        ],
        compiler_params=pltpu.CompilerParams(
            collective_id=0,
            vmem_limit_bytes=64 * 1024 * 1024,
        ),
    )(x, router_W, route_idx, expert_W)
